# baseline (speedup 1.0000x reference)
import functools

import jax
import jax.numpy as jnp
from jax import lax
from jax.experimental import pallas as pl
from jax.experimental.pallas import tpu as pltpu
from jax.experimental.pallas import tpu_sc as plsc

N = 100000
M = 128
C = 20
NGEN = 8
NGRAPHS = 128

NC = 2
NS = 16
NW = NC * NS
CHUNK = 3200
NPAD = NW * CHUNK
ROWS = CHUNK // 128
BINS = NGRAPHS * M
ZSLICE = BINS // NS


TAIL = N - (NW - 1) * CHUNK


def _sc_body(x_hbm, b_hbm, tt_hbm, out_hbm, x_v, b_v, xh_v, idx_v, z_v,
             tt_v, out_v, sem, sem2, hist_s):
    c = lax.axis_index("c")
    s = lax.axis_index("s")
    wid = s * NC + c
    base = wid * CHUNK
    last = wid == NW - 1
    zero = jnp.zeros((16,), jnp.int32)
    one = jnp.ones((16,), jnp.int32)

    tt_d = pltpu.async_copy(tt_hbm, tt_v, sem2)

    @pl.when(jnp.logical_not(last))
    def _():
        descs = [pltpu.async_copy(x_hbm.at[pl.ds(base, CHUNK)], x_v, sem),
                 pltpu.async_copy(b_hbm.at[pl.ds(base, CHUNK)], b_v, sem)]
        for i in range(ZSLICE // 16):
            z_v[pl.ds(i * 16, 16)] = zero
        pltpu.sync_copy(z_v, hist_s.at[pl.ds(s * ZSLICE, ZSLICE)])
        for d in descs:
            d.wait()

    @pl.when(last)
    def _():
        descs = [
            pltpu.async_copy(x_hbm.at[pl.ds(base, TAIL)], x_v.at[pl.ds(0, TAIL)], sem),
            pltpu.async_copy(b_hbm.at[pl.ds(base, TAIL)], b_v.at[pl.ds(0, TAIL)], sem)]
        for i in range(ZSLICE // 16):
            z_v[pl.ds(i * 16, 16)] = zero
        pltpu.sync_copy(z_v, hist_s.at[pl.ds(s * ZSLICE, ZSLICE)])
        for d in descs:
            d.wait()

    gmin = b_v[pl.ds(0, 16)][0]
    gmax = jnp.where(last,
                     b_v[pl.ds(TAIL - 16, 16)][15],
                     b_v[pl.ds(CHUNK - 16, 16)][15])

    def zero_row(r, carry):
        for k in range(8):
            xh_v[pl.ds(r * 128 + k * 16, 16)] = zero
        return carry

    lax.fori_loop(gmin, gmax + 1, zero_row, 0)

    for i in range(TAIL // 16):
        key = b_v[pl.ds(i * 16, 16)] * M + x_v[pl.ds(i * 16, 16)]
        plsc.addupdate_scatter(xh_v, [key], one)

    @pl.when(jnp.logical_not(last))
    def _():
        for i in range(TAIL // 16, CHUNK // 16):
            key = b_v[pl.ds(i * 16, 16)] * M + x_v[pl.ds(i * 16, 16)]
            plsc.addupdate_scatter(xh_v, [key], one)

    plsc.subcore_barrier()

    lanes = lax.iota(jnp.int32, 16)
    kvecs = [lanes + k * 16 for k in range(8)]

    def merge_row(r, carry):
        for k in range(8):
            idx_v[pl.ds(k * 16, 16)] = r * 128 + kvecs[k]
        pltpu.sync_copy(xh_v.at[pl.ds(r * 128, 128)], hist_s.at[idx_v],
                        add=True)
        return carry

    lax.fori_loop(gmin, gmax + 1, merge_row, 0)

    tt_d.wait()
    plsc.subcore_barrier()

    pltpu.sync_copy(hist_s.at[pl.ds(s * ZSLICE, ZSLICE)], z_v)
    for p in range(4):
        acc = jnp.zeros((16,), jnp.float32)
        for half in range(2):
            lg = 2 * p + half
            hf = [z_v[pl.ds(lg * 128 + k * 16, 16)].astype(jnp.float32)
                  for k in range(8)]
            for g in range(NGEN):
                prod = hf[0] * tt_v[pl.ds(g * 128, 16)]
                for k in range(1, 8):
                    prod = prod + hf[k] * tt_v[pl.ds(g * 128 + k * 16, 16)]
                total = jnp.sum(prod)
                lane = half * 8 + g
                mask = lax.iota(jnp.int32, 16) == lane
                acc = jnp.where(mask, -total, acc)
        out_v[pl.ds(p * 16, 16)] = acc
    pltpu.sync_copy(out_v,
                    out_hbm.at[pl.ds(c * NGRAPHS * NGEN + s * 64, 64)])


@jax.jit
def _sc_call(x_p, b_p, tt):
    mesh = plsc.VectorSubcoreMesh(core_axis_name="c", subcore_axis_name="s")
    return pl.kernel(
        _sc_body,
        out_type=jax.ShapeDtypeStruct((NC * NGRAPHS * NGEN,), jnp.float32),
        mesh=mesh,
        compiler_params=pltpu.CompilerParams(needs_layout_passes=False),
        scratch_types=[
            pltpu.VMEM((CHUNK,), jnp.int32),
            pltpu.VMEM((CHUNK,), jnp.int32),
            pltpu.VMEM((NGRAPHS * M,), jnp.int32),
            pltpu.VMEM((128,), jnp.int32),
            pltpu.VMEM((ZSLICE,), jnp.int32),
            pltpu.VMEM((NGEN * M,), jnp.float32),
            pltpu.VMEM((64,), jnp.float32),
            pltpu.SemaphoreType.DMA,
            pltpu.SemaphoreType.DMA,
            pltpu.VMEM_SHARED((BINS,), jnp.int32),
        ],
    )(x_p, b_p, tt)


def _tc_table_body(b3_ref, pi_ref, tt_ref):
    B3 = b3_ref[...]
    Pi = pi_ref[...]
    mB = jnp.max(B3, axis=2, keepdims=True)
    lseB = mB + jnp.log(jnp.sum(jnp.exp(B3 - mB), axis=2, keepdims=True))
    mP = jnp.max(Pi, axis=0, keepdims=True)
    lsePi = mP + jnp.log(jnp.sum(jnp.exp(Pi - mP), axis=0, keepdims=True))
    A = (B3 - lseB) + (Pi - lsePi)[:, :, None]
    numv = jnp.exp(A)
    denom = jnp.sum(numv, axis=0)
    score = jnp.sum(numv * A, axis=0)
    tt_ref[...] = score / denom


def _tc_table(b3, pi):
    return pl.pallas_call(
        _tc_table_body,
        out_shape=jax.ShapeDtypeStruct((NGEN, M), jnp.float32),
    )(b3, pi)


def kernel(x, edge_index, batch, B, Pi):
    del edge_index
    b3 = jnp.transpose(B, (0, 2, 1))
    tt = _tc_table(b3, Pi)
    part = _sc_call(x, batch, tt.reshape(NGEN * M))
    out = part[:NGRAPHS * NGEN] + part[NGRAPHS * NGEN:]
    return out.reshape(NGRAPHS, 1, NGEN)


# speedup vs baseline: 1.0586x; 1.0586x over previous
import functools

import jax
import jax.numpy as jnp
from jax import lax
from jax.experimental import pallas as pl
from jax.experimental.pallas import tpu as pltpu
from jax.experimental.pallas import tpu_sc as plsc

N = 100000
M = 128
C = 20
NGEN = 8
NGRAPHS = 128

NC = 2
NS = 16
NW = NC * NS
CHUNK = 3200
NPAD = NW * CHUNK
ROWS = CHUNK // 128
BINS = NGRAPHS * M
ZSLICE = BINS // NS


TAIL = N - (NW - 1) * CHUNK


def _sc_body(x_hbm, b_hbm, out_hbm, x_v, b_v, xh_v, idx_v, z_v, sem,
             hist_s):
    c = lax.axis_index("c")
    s = lax.axis_index("s")
    wid = s * NC + c
    base = wid * CHUNK
    last = wid == NW - 1
    zero = jnp.zeros((16,), jnp.int32)
    one = jnp.ones((16,), jnp.int32)

    @pl.when(jnp.logical_not(last))
    def _():
        descs = [pltpu.async_copy(x_hbm.at[pl.ds(base, CHUNK)], x_v, sem),
                 pltpu.async_copy(b_hbm.at[pl.ds(base, CHUNK)], b_v, sem)]
        for i in range(ZSLICE // 16):
            z_v[pl.ds(i * 16, 16)] = zero
        pltpu.sync_copy(z_v, hist_s.at[pl.ds(s * ZSLICE, ZSLICE)])
        for d in descs:
            d.wait()

    @pl.when(last)
    def _():
        descs = [
            pltpu.async_copy(x_hbm.at[pl.ds(base, TAIL)], x_v.at[pl.ds(0, TAIL)], sem),
            pltpu.async_copy(b_hbm.at[pl.ds(base, TAIL)], b_v.at[pl.ds(0, TAIL)], sem)]
        for i in range(ZSLICE // 16):
            z_v[pl.ds(i * 16, 16)] = zero
        pltpu.sync_copy(z_v, hist_s.at[pl.ds(s * ZSLICE, ZSLICE)])
        for d in descs:
            d.wait()

    gmin = b_v[pl.ds(0, 16)][0]
    gmax = jnp.where(last,
                     b_v[pl.ds(TAIL - 16, 16)][15],
                     b_v[pl.ds(CHUNK - 16, 16)][15])

    def zero_row(r, carry):
        for k in range(8):
            xh_v[pl.ds(r * 128 + k * 16, 16)] = zero
        return carry

    lax.fori_loop(gmin, gmax + 1, zero_row, 0)

    for i in range(TAIL // 16):
        key = b_v[pl.ds(i * 16, 16)] * M + x_v[pl.ds(i * 16, 16)]
        plsc.addupdate_scatter(xh_v, [key], one)

    @pl.when(jnp.logical_not(last))
    def _():
        for i in range(TAIL // 16, CHUNK // 16):
            key = b_v[pl.ds(i * 16, 16)] * M + x_v[pl.ds(i * 16, 16)]
            plsc.addupdate_scatter(xh_v, [key], one)

    plsc.subcore_barrier()

    lanes = lax.iota(jnp.int32, 16)
    kvecs = [lanes + k * 16 for k in range(8)]

    def merge_row(r, carry):
        for k in range(8):
            idx_v[pl.ds(k * 16, 16)] = r * 128 + kvecs[k]
        pltpu.sync_copy(xh_v.at[pl.ds(r * 128, 128)], hist_s.at[idx_v],
                        add=True)
        return carry

    lax.fori_loop(gmin, gmax + 1, merge_row, 0)

    plsc.subcore_barrier()
    pltpu.sync_copy(hist_s.at[pl.ds(s * ZSLICE, ZSLICE)], z_v)
    pltpu.sync_copy(z_v, out_hbm.at[pl.ds(c * BINS + s * ZSLICE, ZSLICE)])


@jax.jit
def _sc_call(x_p, b_p):
    mesh = plsc.VectorSubcoreMesh(core_axis_name="c", subcore_axis_name="s")
    return pl.kernel(
        _sc_body,
        out_type=jax.ShapeDtypeStruct((NC * BINS,), jnp.int32),
        mesh=mesh,
        compiler_params=pltpu.CompilerParams(needs_layout_passes=False),
        scratch_types=[
            pltpu.VMEM((CHUNK,), jnp.int32),
            pltpu.VMEM((CHUNK,), jnp.int32),
            pltpu.VMEM((NGRAPHS * M,), jnp.int32),
            pltpu.VMEM((128,), jnp.int32),
            pltpu.VMEM((ZSLICE,), jnp.int32),
            pltpu.SemaphoreType.DMA,
            pltpu.VMEM_SHARED((BINS,), jnp.int32),
        ],
    )(x_p, b_p)


def _tc_table_body(b3_ref, pi_ref, tt_ref):
    B3 = b3_ref[...]
    Pi = pi_ref[...]
    mB = jnp.max(B3, axis=2, keepdims=True)
    lseB = mB + jnp.log(jnp.sum(jnp.exp(B3 - mB), axis=2, keepdims=True))
    mP = jnp.max(Pi, axis=0, keepdims=True)
    lsePi = mP + jnp.log(jnp.sum(jnp.exp(Pi - mP), axis=0, keepdims=True))
    A = (B3 - lseB) + (Pi - lsePi)[:, :, None]
    numv = jnp.exp(A)
    denom = jnp.sum(numv, axis=0)
    score = jnp.sum(numv * A, axis=0)
    tt_ref[...] = score / denom


def _tc_table(b3, pi):
    return pl.pallas_call(
        _tc_table_body,
        out_shape=jax.ShapeDtypeStruct((NGEN, M), jnp.float32),
    )(b3, pi)


def _tc_finish_body(tt_ref, h2_ref, out_ref):
    tt = tt_ref[...]
    h2 = h2_ref[...]
    h0 = lax.slice(h2, (0,), (BINS,)).reshape(NGRAPHS, M)
    h1 = lax.slice(h2, (BINS,), (2 * BINS,)).reshape(NGRAPHS, M)
    h = (h0 + h1).astype(jnp.float32)
    res = -lax.dot_general(
        h, tt, (((1,), (1,)), ((), ())),
        preferred_element_type=jnp.float32)
    out_ref[...] = res[:, None, :]


def _tc_finish(tt, h2):
    return pl.pallas_call(
        _tc_finish_body,
        out_shape=jax.ShapeDtypeStruct((NGRAPHS, 1, NGEN), jnp.float32),
    )(tt, h2)


def kernel(x, edge_index, batch, B, Pi):
    del edge_index
    h2 = _sc_call(x, batch)
    b3 = jnp.transpose(B, (0, 2, 1))
    tt = _tc_table(b3, Pi)
    return _tc_finish(tt, h2)


# speedup vs baseline: 1.0591x; 1.0005x over previous
import jax
import jax.numpy as jnp
from jax import lax
from jax.experimental import pallas as pl
from jax.experimental.pallas import tpu as pltpu
from jax.experimental.pallas import tpu_sc as plsc

N = 100000
M = 128
C = 20
NGEN = 8
NGRAPHS = 128

NC = 2
NS = 16
NW = NC * NS
CHUNK = 3200
BINS = NGRAPHS * M
ZSLICE = BINS // NS


TAIL = N - (NW - 1) * CHUNK


def _sc_body(x_hbm, b_hbm, out_hbm, x_v, b_v, xh_v, idx_v, z_v, sem,
             hist_s):
    c = lax.axis_index("c")
    s = lax.axis_index("s")
    wid = s * NC + c
    base = wid * CHUNK
    last = wid == NW - 1
    zero = jnp.zeros((16,), jnp.int32)
    one = jnp.ones((16,), jnp.int32)

    @pl.when(jnp.logical_not(last))
    def _():
        descs = [pltpu.async_copy(x_hbm.at[pl.ds(base, CHUNK)], x_v, sem),
                 pltpu.async_copy(b_hbm.at[pl.ds(base, CHUNK)], b_v, sem)]
        for i in range(ZSLICE // 16):
            z_v[pl.ds(i * 16, 16)] = zero
        pltpu.sync_copy(z_v, hist_s.at[pl.ds(s * ZSLICE, ZSLICE)])
        for d in descs:
            d.wait()

    @pl.when(last)
    def _():
        descs = [
            pltpu.async_copy(x_hbm.at[pl.ds(base, TAIL)], x_v.at[pl.ds(0, TAIL)], sem),
            pltpu.async_copy(b_hbm.at[pl.ds(base, TAIL)], b_v.at[pl.ds(0, TAIL)], sem)]
        for i in range(ZSLICE // 16):
            z_v[pl.ds(i * 16, 16)] = zero
        pltpu.sync_copy(z_v, hist_s.at[pl.ds(s * ZSLICE, ZSLICE)])
        for d in descs:
            d.wait()

    gmin = b_v[pl.ds(0, 16)][0]
    gmax = jnp.where(last,
                     b_v[pl.ds(TAIL - 16, 16)][15],
                     b_v[pl.ds(CHUNK - 16, 16)][15])

    def zero_row(r, carry):
        for k in range(8):
            xh_v[pl.ds(r * 128 + k * 16, 16)] = zero
        return carry

    lax.fori_loop(gmin, gmax + 1, zero_row, 0)

    for i in range(TAIL // 16):
        key = b_v[pl.ds(i * 16, 16)] * M + x_v[pl.ds(i * 16, 16)]
        plsc.addupdate_scatter(xh_v, [key], one)

    @pl.when(jnp.logical_not(last))
    def _():
        for i in range(TAIL // 16, CHUNK // 16):
            key = b_v[pl.ds(i * 16, 16)] * M + x_v[pl.ds(i * 16, 16)]
            plsc.addupdate_scatter(xh_v, [key], one)

    plsc.subcore_barrier()

    lanes = lax.iota(jnp.int32, 16)
    kvecs = [lanes + k * 16 for k in range(8)]

    def merge_row(r, carry):
        for k in range(8):
            idx_v[pl.ds(k * 16, 16)] = r * 128 + kvecs[k]
        pltpu.sync_copy(xh_v.at[pl.ds(r * 128, 128)], hist_s.at[idx_v],
                        add=True)
        return carry

    lax.fori_loop(gmin, gmax + 1, merge_row, 0)

    plsc.subcore_barrier()
    pltpu.sync_copy(hist_s.at[pl.ds(s * ZSLICE, ZSLICE)], z_v)
    pltpu.sync_copy(z_v, out_hbm.at[pl.ds(c * BINS + s * ZSLICE, ZSLICE)])


@jax.jit
def _sc_call(x_p, b_p):
    mesh = plsc.VectorSubcoreMesh(core_axis_name="c", subcore_axis_name="s")
    return pl.kernel(
        _sc_body,
        out_type=jax.ShapeDtypeStruct((NC * BINS,), jnp.int32),
        mesh=mesh,
        compiler_params=pltpu.CompilerParams(needs_layout_passes=False),
        scratch_types=[
            pltpu.VMEM((CHUNK,), jnp.int32),
            pltpu.VMEM((CHUNK,), jnp.int32),
            pltpu.VMEM((NGRAPHS * M,), jnp.int32),
            pltpu.VMEM((128,), jnp.int32),
            pltpu.VMEM((ZSLICE,), jnp.int32),
            pltpu.SemaphoreType.DMA,
            pltpu.VMEM_SHARED((BINS,), jnp.int32),
        ],
    )(x_p, b_p)


def _tc_table_body(b3_ref, pi_ref, tt_ref):
    B3 = b3_ref[...]
    Pi = pi_ref[...]
    mB = jnp.max(B3, axis=2, keepdims=True)
    lseB = mB + jnp.log(jnp.sum(jnp.exp(B3 - mB), axis=2, keepdims=True))
    mP = jnp.max(Pi, axis=0, keepdims=True)
    lsePi = mP + jnp.log(jnp.sum(jnp.exp(Pi - mP), axis=0, keepdims=True))
    A = (B3 - lseB) + (Pi - lsePi)[:, :, None]
    numv = jnp.exp(A)
    denom = jnp.sum(numv, axis=0)
    score = jnp.sum(numv * A, axis=0)
    tt_ref[...] = score / denom


def _tc_table(b3, pi):
    return pl.pallas_call(
        _tc_table_body,
        out_shape=jax.ShapeDtypeStruct((NGEN, M), jnp.float32),
    )(b3, pi)


def _tc_finish_body(tt_ref, h2_ref, out_ref):
    tt = tt_ref[...]
    h2 = h2_ref[...]
    h0 = lax.slice(h2, (0,), (BINS,)).reshape(NGRAPHS, M)
    h1 = lax.slice(h2, (BINS,), (2 * BINS,)).reshape(NGRAPHS, M)
    h = (h0 + h1).astype(jnp.float32)
    res = -lax.dot_general(
        h, tt, (((1,), (1,)), ((), ())),
        preferred_element_type=jnp.float32)
    out_ref[...] = res[:, None, :]


def _tc_finish(tt, h2):
    return pl.pallas_call(
        _tc_finish_body,
        out_shape=jax.ShapeDtypeStruct((NGRAPHS, 1, NGEN), jnp.float32),
    )(tt, h2)


def kernel(x, edge_index, batch, B, Pi):
    del edge_index
    h2 = _sc_call(x, batch)
    b3 = jnp.transpose(B, (0, 2, 1))
    tt = _tc_table(b3, Pi)
    return _tc_finish(tt, h2)
